# baseline (speedup 1.0000x reference)
import functools

import jax
import jax.numpy as jnp
from jax import lax
from jax.experimental import pallas as pl
from jax.experimental.pallas import tpu as pltpu
from jax.experimental.pallas import tpu_sc as plsc

NC = 2
NS = 16
NW = NC * NS
LANES = 16
GATHER_CHUNK = 128


def _sc_embed(ids, tok_table, pos_table, *, seq_len, n_dim):
    rows_per_w = seq_len // NW
    n_chunks = rows_per_w // GATHER_CHUNK
    mesh = plsc.VectorSubcoreMesh(core_axis_name="c", subcore_axis_name="s")

    @functools.partial(
        pl.kernel,
        mesh=mesh,
        compiler_params=pltpu.CompilerParams(use_tc_tiling_on_sc=False),
        out_type=jax.ShapeDtypeStruct((seq_len, n_dim), jnp.float32),
        scratch_types=[
            pltpu.VMEM((rows_per_w,), jnp.int32),
            pltpu.VMEM((rows_per_w, n_dim), jnp.float32),
            pltpu.VMEM((rows_per_w, n_dim), jnp.float32),
            pltpu.SemaphoreType.DMA,
            pltpu.SemaphoreType.DMA,
        ],
    )
    def k(ids_hbm, tok_hbm, pos_hbm, out_hbm, idx_v, rows_v, pos_v, gsem,
          psem):
        wid = lax.axis_index("s") * NC + lax.axis_index("c")
        base = wid * rows_per_w

        pltpu.sync_copy(ids_hbm.at[pl.ds(base, rows_per_w)], idx_v)

        pos_cp = pltpu.async_copy(
            pos_hbm.at[pl.ds(base, rows_per_w)], pos_v, psem)

        gathers = []
        for c in range(n_chunks):
            gathers.append(pltpu.async_copy(
                tok_hbm.at[idx_v.at[pl.ds(c * GATHER_CHUNK, GATHER_CHUNK)]],
                rows_v.at[pl.ds(c * GATHER_CHUNK, GATHER_CHUNK)],
                gsem))
        for cp in gathers:
            cp.wait()
        pos_cp.wait()

        @pl.loop(0, rows_per_w)
        def _(r):
            for c0 in range(0, n_dim, LANES):
                sl = (r, pl.ds(c0, LANES))
                rows_v[sl] = rows_v[sl] + pos_v[sl]

        pltpu.sync_copy(rows_v, out_hbm.at[pl.ds(base, rows_per_w)])

    return k(ids, tok_table, pos_table)


def kernel(ids, tok_table, pos_table):
    seq_len = ids.shape[0]
    n_dim = tok_table.shape[1]
    out = _sc_embed(ids.astype(jnp.int32), tok_table, pos_table,
                    seq_len=seq_len, n_dim=n_dim)
    return out[None]


# speedup vs baseline: 1.7012x; 1.7012x over previous
import dataclasses
import functools

import jax
import jax.numpy as jnp
from jax import lax
from jax.experimental import pallas as pl
from jax.experimental.pallas import tpu as pltpu
from jax.experimental.pallas import tpu_sc as plsc

NC = 2
NS = 16
NW = NC * NS
LANES = 16


def _sc_gather(ids, tok_table, *, seq_len, n_dim):
    rows_per_w = seq_len // NW
    n_groups = rows_per_w // LANES
    mesh = plsc.VectorSubcoreMesh(core_axis_name="c", subcore_axis_name="s")

    cp = pltpu.CompilerParams()
    if "needs_layout_passes" in pltpu.CompilerParams.__dataclass_fields__:
        cp = dataclasses.replace(cp, needs_layout_passes=False)

    @functools.partial(
        pl.kernel,
        mesh=mesh,
        compiler_params=cp,
        out_type=jax.ShapeDtypeStruct((seq_len, n_dim), jnp.float32),
        scratch_types=[
            pltpu.VMEM((rows_per_w,), jnp.int32),
            pltpu.VMEM((rows_per_w, n_dim), jnp.float32),
            pltpu.SemaphoreType.DMA,
        ],
    )
    def k(ids_hbm, tok_hbm, out_hbm, idx_v, rows_v, gsem):
        wid = lax.axis_index("s") * NC + lax.axis_index("c")
        base = wid * rows_per_w

        pltpu.sync_copy(ids_hbm.at[pl.ds(base, rows_per_w)], idx_v)

        lane = jnp.arange(LANES, dtype=jnp.int32)

        @pl.loop(0, n_groups)
        def _(g):
            vec = idx_v[pl.ds(g * LANES, LANES)]
            for l in range(LANES):
                i = jnp.sum(jnp.where(lane == l, vec, 0))
                pltpu.async_copy(
                    tok_hbm.at[pl.ds(i, 1)],
                    rows_v.at[pl.ds(g * LANES + l, 1)],
                    gsem)

        pltpu.make_async_copy(
            tok_hbm.at[pl.ds(0, rows_per_w)], rows_v, gsem).wait()

        pltpu.sync_copy(rows_v, out_hbm.at[pl.ds(base, rows_per_w)])

    return k(ids, tok_table)


def _tc_add(a, b):
    def body(a_ref, b_ref, o_ref):
        o_ref[...] = a_ref[...] + b_ref[...]

    return pl.pallas_call(
        body,
        out_shape=jax.ShapeDtypeStruct(a.shape, a.dtype),
    )(a, b)


def kernel(ids, tok_table, pos_table):
    seq_len = ids.shape[0]
    n_dim = tok_table.shape[1]
    gathered = _sc_gather(ids.astype(jnp.int32), tok_table,
                          seq_len=seq_len, n_dim=n_dim)
    out = _tc_add(gathered, pos_table)
    return out[None]
